# baseline (speedup 1.0000x reference)
import functools

import jax
import jax.numpy as jnp
from jax import lax
from jax.experimental import pallas as pl
from jax.experimental.pallas import tpu as pltpu
from jax.experimental.pallas import tpu_sc as plsc

_D = 512
_SEGS = 512
_SEG_LEN = 32
_TOTAL = _SEGS * _SEG_LEN
_LANES = 16

_NW = 32
_ELEMS = _D * _TOTAL
_OUT = _D * _SEGS
_ELEMS_W = _ELEMS // _NW
_OUT_W = _OUT // _NW
_NCHUNK = 8
_CHUNK = _ELEMS_W // _NCHUNK
_GROUPS = _CHUNK // (_LANES * _SEG_LEN)


def _make_sc_kernel():
    info = plsc.get_sparse_core_info()
    nc = info.num_cores
    mesh = plsc.VectorSubcoreMesh(core_axis_name="c", subcore_axis_name="s")

    @functools.partial(
        pl.kernel,
        out_type=jax.ShapeDtypeStruct((_OUT,), jnp.float32),
        mesh=mesh,
        scratch_types=[
            pltpu.VMEM((_CHUNK,), jnp.float32),
            pltpu.VMEM((_CHUNK,), jnp.float32),
            pltpu.VMEM((_OUT_W,), jnp.float32),
            pltpu.SemaphoreType.DMA,
            pltpu.SemaphoreType.DMA,
        ],
        compiler_params=pltpu.CompilerParams(needs_layout_passes=False),
    )
    def prod_at(x_hbm, out_hbm, buf0, buf1, out_v, sem0, sem1):
        wid = lax.axis_index("s") * nc + lax.axis_index("c")
        in_base = wid * _ELEMS_W
        bufs = (buf0, buf1)
        sems = (sem0, sem1)
        stride_iota = lax.broadcasted_iota(jnp.int32, (_LANES,), 0) * _SEG_LEN

        handles = [None, None]
        handles[0] = pltpu.async_copy(
            x_hbm.at[pl.ds(in_base, _CHUNK)], buf0, sem0)
        for c in range(_NCHUNK):
            b = c % 2
            if c + 1 < _NCHUNK:
                handles[1 - b] = pltpu.async_copy(
                    x_hbm.at[pl.ds(in_base + (c + 1) * _CHUNK, _CHUNK)],
                    bufs[1 - b], sems[1 - b])
            handles[b].wait()
            buf = bufs[b]
            out_off = c * (_GROUPS * _LANES)

            def grp_body(g, carry, buf=buf, out_off=out_off):
                base = g * (_LANES * _SEG_LEN)
                acc = plsc.load_gather(buf, [stride_iota + base])
                for i in range(1, _SEG_LEN):
                    acc = acc * plsc.load_gather(buf, [stride_iota + (base + i)])
                out_v[pl.ds(out_off + g * _LANES, _LANES)] = acc
                return carry

            lax.fori_loop(0, _GROUPS, grp_body, 0)

        pltpu.sync_copy(out_v, out_hbm.at[pl.ds(wid * _OUT_W, _OUT_W)])

    return prod_at


_sc_kernel = _make_sc_kernel()


def kernel(x):
    out_flat = _sc_kernel(x.reshape(_ELEMS))
    return out_flat.reshape(_D, _SEGS)


# speedup vs baseline: 2.0675x; 2.0675x over previous
import functools

import jax
import jax.numpy as jnp
from jax import lax
from jax.experimental import pallas as pl
from jax.experimental.pallas import tpu as pltpu
from jax.experimental.pallas import tpu_sc as plsc

_D = 512
_SEGS = 512
_SEG_LEN = 32
_TOTAL = _SEGS * _SEG_LEN
_LANES = 16

_NW = 32
_ELEMS = _D * _TOTAL
_OUT = _D * _SEGS
_ELEMS_W = _ELEMS // _NW
_OUT_W = _OUT // _NW
_NCHUNK = 8
_CHUNK = _ELEMS_W // _NCHUNK
_GROUPS = _CHUNK // (_LANES * _SEG_LEN)


def _make_sc_kernel():
    info = plsc.get_sparse_core_info()
    nc = info.num_cores
    mesh = plsc.VectorSubcoreMesh(core_axis_name="c", subcore_axis_name="s")

    @functools.partial(
        pl.kernel,
        out_type=jax.ShapeDtypeStruct((_OUT,), jnp.float32),
        mesh=mesh,
        scratch_types=[
            pltpu.VMEM((_CHUNK,), jnp.float32),
            pltpu.VMEM((_CHUNK,), jnp.float32),
            pltpu.VMEM((_OUT_W,), jnp.float32),
            pltpu.SemaphoreType.DMA,
            pltpu.SemaphoreType.DMA,
        ],
        compiler_params=pltpu.CompilerParams(needs_layout_passes=False),
    )
    def prod_at(x_hbm, out_hbm, buf0, buf1, out_v, sem0, sem1):
        wid = lax.axis_index("s") * nc + lax.axis_index("c")
        in_base = wid * _ELEMS_W
        bufs = (buf0, buf1)
        sems = (sem0, sem1)
        lane_iota = lax.broadcasted_iota(jnp.int32, (_LANES,), 0)
        rot_idx = [
            lane_iota * _SEG_LEN + ((lane_iota + i) & (_SEG_LEN - 1))
            for i in range(_SEG_LEN)
        ]

        handles = [None, None]
        handles[0] = pltpu.async_copy(
            x_hbm.at[pl.ds(in_base, _CHUNK)], buf0, sem0)
        for c in range(_NCHUNK):
            b = c % 2
            if c + 1 < _NCHUNK:
                handles[1 - b] = pltpu.async_copy(
                    x_hbm.at[pl.ds(in_base + (c + 1) * _CHUNK, _CHUNK)],
                    bufs[1 - b], sems[1 - b])
            handles[b].wait()
            buf = bufs[b]
            out_off = c * (_GROUPS * _LANES)

            def grp_body(g, carry, buf=buf, out_off=out_off):
                base = g * (_LANES * _SEG_LEN)
                acc = plsc.load_gather(buf, [rot_idx[0] + base])
                for i in range(1, _SEG_LEN):
                    acc = acc * plsc.load_gather(buf, [rot_idx[i] + base])
                out_v[pl.ds(out_off + g * _LANES, _LANES)] = acc
                return carry

            lax.fori_loop(0, _GROUPS, grp_body, 0)

        pltpu.sync_copy(out_v, out_hbm.at[pl.ds(wid * _OUT_W, _OUT_W)])

    return prod_at


_sc_kernel = _make_sc_kernel()


def kernel(x):
    out_flat = _sc_kernel(x.reshape(_ELEMS))
    return out_flat.reshape(_D, _SEGS)


# speedup vs baseline: 2.2756x; 1.1006x over previous
import functools

import jax
import jax.numpy as jnp
from jax import lax
from jax.experimental import pallas as pl
from jax.experimental.pallas import tpu as pltpu
from jax.experimental.pallas import tpu_sc as plsc

_D = 512
_SEGS = 512
_SEG_LEN = 32
_TOTAL = _SEGS * _SEG_LEN
_LANES = 16

_NW = 32
_ELEMS = _D * _TOTAL
_OUT = _D * _SEGS
_ELEMS_W = _ELEMS // _NW
_OUT_W = _OUT // _NW
_NCHUNK = 8
_CHUNK = _ELEMS_W // _NCHUNK
_GROUPS = _CHUNK // (_LANES * _SEG_LEN)


def _make_sc_kernel():
    info = plsc.get_sparse_core_info()
    nc = info.num_cores
    mesh = plsc.VectorSubcoreMesh(core_axis_name="c", subcore_axis_name="s")

    @functools.partial(
        pl.kernel,
        out_type=jax.ShapeDtypeStruct((_OUT,), jnp.float32),
        mesh=mesh,
        scratch_types=[
            pltpu.VMEM((_CHUNK,), jnp.float32),
            pltpu.VMEM((_CHUNK,), jnp.float32),
            pltpu.VMEM((_OUT_W,), jnp.float32),
            pltpu.SemaphoreType.DMA,
            pltpu.SemaphoreType.DMA,
        ],
        compiler_params=pltpu.CompilerParams(needs_layout_passes=False),
    )
    def prod_at(x_hbm, out_hbm, buf0, buf1, out_v, sem0, sem1):
        wid = lax.axis_index("s") * nc + lax.axis_index("c")
        in_base = wid * _ELEMS_W
        bufs = (buf0, buf1)
        sems = (sem0, sem1)
        lane_iota = lax.broadcasted_iota(jnp.int32, (_LANES,), 0)
        rot_idx = [
            lane_iota * _SEG_LEN + ((lane_iota + i) & (_SEG_LEN - 1))
            for i in range(_SEG_LEN)
        ]

        handles = [None, None]
        handles[0] = pltpu.async_copy(
            x_hbm.at[pl.ds(in_base, _CHUNK)], buf0, sem0)
        for c in range(_NCHUNK):
            b = c % 2
            if c + 1 < _NCHUNK:
                handles[1 - b] = pltpu.async_copy(
                    x_hbm.at[pl.ds(in_base + (c + 1) * _CHUNK, _CHUNK)],
                    bufs[1 - b], sems[1 - b])
            handles[b].wait()
            buf = bufs[b]
            out_off = c * (_GROUPS * _LANES)

            def grp_body(g, carry, buf=buf, out_off=out_off):
                base = g * (_LANES * _SEG_LEN)
                vals = [
                    plsc.load_gather(buf, [rot_idx[i] + base])
                    for i in range(_SEG_LEN)
                ]
                while len(vals) > 1:
                    vals = [a * b for a, b in zip(vals[::2], vals[1::2])]
                acc = vals[0]
                out_v[pl.ds(out_off + g * _LANES, _LANES)] = acc
                return carry

            lax.fori_loop(0, _GROUPS, grp_body, 0)

        pltpu.sync_copy(out_v, out_hbm.at[pl.ds(wid * _OUT_W, _OUT_W)])

    return prod_at


_sc_kernel = _make_sc_kernel()


def kernel(x):
    out_flat = _sc_kernel(x.reshape(_ELEMS))
    return out_flat.reshape(_D, _SEGS)
